# baseline (speedup 1.0000x reference)
import jax
import jax.numpy as jnp
from jax import lax
from jax.experimental import pallas as pl
from jax.experimental.pallas import tpu as pltpu
from jax.experimental.pallas import tpu_sc as plsc

G = 16
C = 32
RES = (128, 64, 32, 16)
N = 8192
B = 2
NW = 32
CHUNK = 64
NCHUNK = N // CHUNK
P = B * G * N


def _round_bf16(x):
    xi = plsc.bitcast(x, jnp.int32)
    rounded = (xi + 0x7FFF + (lax.shift_right_logical(xi, 16) & 1)) & jnp.int32(-65536)
    return plsc.bitcast(rounded, jnp.float32)


def _sc_body(pts_hbm, proj_hbm, t0, t1, f2_hbm, f3_hbm, out_hbm,
             pts_v, proj_v, idx_v, w_v, rows_v, out_v, tab2_v, tab3_v,
             gsem0, gsem1, osem0, osem1):
    wid = lax.axis_index("s") * 2 + lax.axis_index("c")
    g_id = lax.rem(wid, G)
    tables = (t0, t1)
    gsems = (gsem0, gsem1)
    osems = (osem0, osem1)

    pltpu.sync_copy(pts_hbm.at[wid], pts_v)
    pltpu.sync_copy(proj_hbm.at[wid], proj_v)
    pltpu.sync_copy(f2_hbm.at[g_id], tab2_v)
    pltpu.sync_copy(f3_hbm.at[g_id], tab3_v)
    pv = _round_bf16(proj_v[...])
    a0 = pv[0]
    a1 = pv[1]
    a2 = pv[2]
    b0 = pv[3]
    b1 = pv[4]
    b2 = pv[5]
    lanes = lax.iota(jnp.int32, 16)
    cstride2 = lanes * (32 * 32)
    cstride3 = lanes * (16 * 16)

    def compute_chunk(ci, slot):
        base = ci * CHUNK

        def group_body(gi, c2):
            off = base + gi * 16
            loc = gi * 16
            px = _round_bf16(pts_v[0, pl.ds(off, 16)])
            py = _round_bf16(pts_v[1, pl.ds(off, 16)])
            pz = _round_bf16(pts_v[2, pl.ds(off, 16)])
            u = px * a0 + py * a1 + pz * a2
            v = px * b0 + py * b1 + pz * b2
            for l, r in enumerate(RES):
                ix = jnp.clip((u + 1.0) * 0.5 * (r - 1), 0.0, float(r - 1))
                iy = jnp.clip((v + 1.0) * 0.5 * (r - 1), 0.0, float(r - 1))
                x0 = ix.astype(jnp.int32)
                y0 = iy.astype(jnp.int32)
                wx = ix - x0.astype(jnp.float32)
                wy = iy - y0.astype(jnp.float32)
                x1 = jnp.minimum(x0 + 1, r - 1)
                y1 = jnp.minimum(y0 + 1, r - 1)
                goff = g_id * (r * r) if l < 2 else 0
                rbase = goff + y0 * r
                rbase1 = goff + y1 * r
                ibase = slot * 16 * CHUNK
                idx_v[pl.ds(ibase + (4 * l + 0) * CHUNK + loc, 16)] = rbase + x0
                idx_v[pl.ds(ibase + (4 * l + 1) * CHUNK + loc, 16)] = rbase + x1
                idx_v[pl.ds(ibase + (4 * l + 2) * CHUNK + loc, 16)] = rbase1 + x0
                idx_v[pl.ds(ibase + (4 * l + 3) * CHUNK + loc, 16)] = rbase1 + x1
                wbase = slot * 16 * CHUNK
                w_v[pl.ds(wbase + (4 * l + 0) * CHUNK + loc, 16)] = (1.0 - wx) * (1.0 - wy)
                w_v[pl.ds(wbase + (4 * l + 1) * CHUNK + loc, 16)] = wx * (1.0 - wy)
                w_v[pl.ds(wbase + (4 * l + 2) * CHUNK + loc, 16)] = (1.0 - wx) * wy
                w_v[pl.ds(wbase + (4 * l + 3) * CHUNK + loc, 16)] = wx * wy
            return c2

        lax.fori_loop(0, CHUNK // 16, group_body, 0)

    def fire_chunk(slot):
        for l in range(2):
            for cnr in range(4):
                lc = 4 * l + cnr
                idx_slice = idx_v.at[pl.ds((slot * 16 + lc) * CHUNK, CHUNK)]
                pltpu.async_copy(tables[l].at[idx_slice],
                                 rows_v.at[slot, lc], gsems[slot])

    def wait_chunk(slot):
        for l in range(2):
            for cnr in range(4):
                lc = 4 * l + cnr
                idx_slice = idx_v.at[pl.ds((slot * 16 + lc) * CHUNK, CHUNK)]
                pltpu.make_async_copy(tables[l].at[idx_slice],
                                      rows_v.at[slot, lc], gsems[slot]).wait()

    def acc_chunk(ci, slot):
        wbase = slot * 16 * CHUNK
        ibase = slot * 16 * CHUNK

        def point_acc(p):
            wv = plsc.load_gather(w_v, [wbase + lanes * CHUNK + p])
            iv = plsc.load_gather(idx_v, [ibase + lanes * CHUNK + p])
            acc0 = jnp.zeros((16,), jnp.float32)
            acc1 = jnp.zeros((16,), jnp.float32)
            for lc in range(8):
                wt = wv[lc]
                acc0 = acc0 + wt * rows_v[slot, lc, p, pl.ds(0, 16)]
                acc1 = acc1 + wt * rows_v[slot, lc, p, pl.ds(16, 16)]
            for lc in range(8, 12):
                wt = wv[lc]
                col = cstride2 + iv[lc]
                acc0 = acc0 + wt * plsc.load_gather(tab2_v, [col])
                acc1 = acc1 + wt * plsc.load_gather(tab2_v, [col + 16 * 32 * 32])
            for lc in range(12, 16):
                wt = wv[lc]
                col = cstride3 + iv[lc]
                acc0 = acc0 + wt * plsc.load_gather(tab3_v, [col])
                acc1 = acc1 + wt * plsc.load_gather(tab3_v, [col + 16 * 16 * 16])
            out_v[slot, p, pl.ds(0, 16)] = acc0
            out_v[slot, p, pl.ds(16, 16)] = acc1

        def acc_body(k, c2):
            point_acc(2 * k)
            point_acc(2 * k + 1)
            return c2

        lax.fori_loop(0, CHUNK // 2, acc_body, 0)
        pltpu.async_copy(out_v.at[slot],
                         out_hbm.at[pl.ds(wid * N + ci * CHUNK, CHUNK)],
                         osems[slot])

    compute_chunk(0, 0)
    fire_chunk(0)

    def ring_body(k, carry):
        for b in range(2):
            ci = 2 * k + b
            nxt = ci + 1
            slot = b
            other = 1 - b

            @pl.when(nxt < NCHUNK)
            def _():
                compute_chunk(nxt, other)
                fire_chunk(other)

            wait_chunk(slot)

            @pl.when(ci >= 2)
            def _():
                pltpu.make_async_copy(
                    out_v.at[slot],
                    out_hbm.at[pl.ds(wid * N + (ci - 2) * CHUNK, CHUNK)],
                    osems[slot]).wait()

            acc_chunk(ci, slot)
        return carry

    lax.fori_loop(0, NCHUNK // 2, ring_body, 0)

    for slot, ci in ((0, NCHUNK - 2), (1, NCHUNK - 1)):
        pltpu.make_async_copy(
            out_v.at[slot],
            out_hbm.at[pl.ds(wid * N + ci * CHUNK, CHUNK)],
            osems[slot]).wait()


_sc_call = pl.kernel(
    _sc_body,
    out_type=jax.ShapeDtypeStruct((P, C), jnp.float32),
    mesh=plsc.VectorSubcoreMesh(core_axis_name="c", subcore_axis_name="s"),
    scratch_types=[
        pltpu.VMEM((3, N), jnp.float32),
        pltpu.VMEM((16,), jnp.float32),
        pltpu.VMEM((2 * 16 * CHUNK,), jnp.int32),
        pltpu.VMEM((2 * 16 * CHUNK,), jnp.float32),
        pltpu.VMEM((2, 8, CHUNK, C), jnp.float32),
        pltpu.VMEM((2, CHUNK, C), jnp.float32),
        pltpu.VMEM((C * 32 * 32,), jnp.float32),
        pltpu.VMEM((C * 16 * 16,), jnp.float32),
        pltpu.SemaphoreType.DMA,
        pltpu.SemaphoreType.DMA,
        pltpu.SemaphoreType.DMA,
        pltpu.SemaphoreType.DMA,
    ],
    compiler_params=pltpu.CompilerParams(
        needs_layout_passes=False, use_tc_tiling_on_sc=False),
)


@jax.jit
def kernel(points, node_uv_projection, featmap0, featmap1, featmap2, featmap3):
    pts_r = points.transpose(0, 1, 3, 2).reshape(NW, 3, N)
    proj_pad = jnp.pad(node_uv_projection.reshape(NW, 6), ((0, 0), (0, 10)))
    tabs = [fm.transpose(0, 2, 3, 1).reshape(G * r * r, C)
            for fm, r in zip((featmap0, featmap1), RES[:2])]
    f2 = featmap2.reshape(G, C * 32 * 32)
    f3 = featmap3.reshape(G, C * 16 * 16)
    out = _sc_call(pts_r, proj_pad, tabs[0], tabs[1], f2, f3)
    return out.reshape(B, G, N, C)


# speedup vs baseline: 2.4527x; 2.4527x over previous
import jax
import jax.numpy as jnp
from jax import lax
from jax.experimental import pallas as pl
from jax.experimental.pallas import tpu as pltpu
from jax.experimental.pallas import tpu_sc as plsc

G = 16
C = 32
RES = (128, 64, 32, 16)
N = 8192
B = 2
NW = 32
CHUNK = 64
NCHUNK = N // CHUNK
P = B * G * N


def _round_bf16(x):
    xi = plsc.bitcast(x, jnp.int32)
    rounded = (xi + 0x7FFF + (lax.shift_right_logical(xi, 16) & 1)) & jnp.int32(-65536)
    return plsc.bitcast(rounded, jnp.float32)


def _sc_body(pts_hbm, proj_hbm, t0, t1, f2_hbm, f3_hbm, out_hbm,
             pts_v, proj_v, idx_v, w_v, rows_v, out_v, tab2_v, tab3_v,
             gsem0, gsem1, osem0, osem1):
    wid = lax.axis_index("s") * 2 + lax.axis_index("c")
    g_id = lax.rem(wid, G)
    tables = (t0, t1)
    gsems = (gsem0, gsem1)
    osems = (osem0, osem1)

    pltpu.sync_copy(pts_hbm.at[wid], pts_v)
    pltpu.sync_copy(proj_hbm.at[wid], proj_v)
    pltpu.sync_copy(f2_hbm.at[pl.ds(g_id * (32 * 32), 32 * 32)], tab2_v)
    pltpu.sync_copy(f3_hbm.at[pl.ds(g_id * (16 * 16), 16 * 16)], tab3_v)
    pv = _round_bf16(proj_v[...])
    a0 = pv[0]
    a1 = pv[1]
    a2 = pv[2]
    b0 = pv[3]
    b1 = pv[4]
    b2 = pv[5]
    lanes = lax.iota(jnp.int32, 16)

    def compute_chunk(ci, slot):
        base = ci * CHUNK

        def group_body(gi, c2):
            off = base + gi * 16
            loc = gi * 16
            px = _round_bf16(pts_v[0, pl.ds(off, 16)])
            py = _round_bf16(pts_v[1, pl.ds(off, 16)])
            pz = _round_bf16(pts_v[2, pl.ds(off, 16)])
            u = px * a0 + py * a1 + pz * a2
            v = px * b0 + py * b1 + pz * b2
            for l, r in enumerate(RES):
                ix = jnp.clip((u + 1.0) * 0.5 * (r - 1), 0.0, float(r - 1))
                iy = jnp.clip((v + 1.0) * 0.5 * (r - 1), 0.0, float(r - 1))
                x0 = ix.astype(jnp.int32)
                y0 = iy.astype(jnp.int32)
                wx = ix - x0.astype(jnp.float32)
                wy = iy - y0.astype(jnp.float32)
                x1 = jnp.minimum(x0 + 1, r - 1)
                y1 = jnp.minimum(y0 + 1, r - 1)
                goff = g_id * (r * r) if l < 2 else 0
                rbase = goff + y0 * r
                rbase1 = goff + y1 * r
                ibase = slot * 16 * CHUNK
                idx_v[pl.ds(ibase + (4 * l + 0) * CHUNK + loc, 16)] = rbase + x0
                idx_v[pl.ds(ibase + (4 * l + 1) * CHUNK + loc, 16)] = rbase + x1
                idx_v[pl.ds(ibase + (4 * l + 2) * CHUNK + loc, 16)] = rbase1 + x0
                idx_v[pl.ds(ibase + (4 * l + 3) * CHUNK + loc, 16)] = rbase1 + x1
                wbase = slot * 16 * CHUNK
                w_v[pl.ds(wbase + (4 * l + 0) * CHUNK + loc, 16)] = (1.0 - wx) * (1.0 - wy)
                w_v[pl.ds(wbase + (4 * l + 1) * CHUNK + loc, 16)] = wx * (1.0 - wy)
                w_v[pl.ds(wbase + (4 * l + 2) * CHUNK + loc, 16)] = (1.0 - wx) * wy
                w_v[pl.ds(wbase + (4 * l + 3) * CHUNK + loc, 16)] = wx * wy
            return c2

        lax.fori_loop(0, CHUNK // 16, group_body, 0)

    def fire_chunk(slot):
        for l in range(2):
            for cnr in range(4):
                lc = 4 * l + cnr
                idx_slice = idx_v.at[pl.ds((slot * 16 + lc) * CHUNK, CHUNK)]
                pltpu.async_copy(tables[l].at[idx_slice],
                                 rows_v.at[slot, lc], gsems[slot])

    def wait_chunk(slot):
        for l in range(2):
            for cnr in range(4):
                lc = 4 * l + cnr
                idx_slice = idx_v.at[pl.ds((slot * 16 + lc) * CHUNK, CHUNK)]
                pltpu.make_async_copy(tables[l].at[idx_slice],
                                      rows_v.at[slot, lc], gsems[slot]).wait()

    def acc_chunk(ci, slot):
        wbase = slot * 16 * CHUNK
        ibase = slot * 16 * CHUNK

        def point_acc(p):
            wv = plsc.load_gather(w_v, [wbase + lanes * CHUNK + p])
            iv = plsc.load_gather(idx_v, [ibase + lanes * CHUNK + p])
            acc0 = jnp.zeros((16,), jnp.float32)
            acc1 = jnp.zeros((16,), jnp.float32)
            for lc in range(8):
                wt = wv[lc]
                acc0 = acc0 + wt * rows_v[slot, lc, p, pl.ds(0, 16)]
                acc1 = acc1 + wt * rows_v[slot, lc, p, pl.ds(16, 16)]
            for lc in range(8, 12):
                wt = wv[lc]
                row = iv[lc]
                acc0 = acc0 + wt * tab2_v[row, pl.ds(0, 16)]
                acc1 = acc1 + wt * tab2_v[row, pl.ds(16, 16)]
            for lc in range(12, 16):
                wt = wv[lc]
                row = iv[lc]
                acc0 = acc0 + wt * tab3_v[row, pl.ds(0, 16)]
                acc1 = acc1 + wt * tab3_v[row, pl.ds(16, 16)]
            out_v[slot, p, pl.ds(0, 16)] = acc0
            out_v[slot, p, pl.ds(16, 16)] = acc1

        def acc_body(k, c2):
            point_acc(2 * k)
            point_acc(2 * k + 1)
            return c2

        lax.fori_loop(0, CHUNK // 2, acc_body, 0)
        pltpu.async_copy(out_v.at[slot],
                         out_hbm.at[pl.ds(wid * N + ci * CHUNK, CHUNK)],
                         osems[slot])

    compute_chunk(0, 0)
    fire_chunk(0)

    def ring_body(k, carry):
        for b in range(2):
            ci = 2 * k + b
            nxt = ci + 1
            slot = b
            other = 1 - b

            @pl.when(nxt < NCHUNK)
            def _():
                compute_chunk(nxt, other)
                fire_chunk(other)

            wait_chunk(slot)

            @pl.when(ci >= 2)
            def _():
                pltpu.make_async_copy(
                    out_v.at[slot],
                    out_hbm.at[pl.ds(wid * N + (ci - 2) * CHUNK, CHUNK)],
                    osems[slot]).wait()

            acc_chunk(ci, slot)
        return carry

    lax.fori_loop(0, NCHUNK // 2, ring_body, 0)

    for slot, ci in ((0, NCHUNK - 2), (1, NCHUNK - 1)):
        pltpu.make_async_copy(
            out_v.at[slot],
            out_hbm.at[pl.ds(wid * N + ci * CHUNK, CHUNK)],
            osems[slot]).wait()


_sc_call = pl.kernel(
    _sc_body,
    out_type=jax.ShapeDtypeStruct((P, C), jnp.float32),
    mesh=plsc.VectorSubcoreMesh(core_axis_name="c", subcore_axis_name="s"),
    scratch_types=[
        pltpu.VMEM((3, N), jnp.float32),
        pltpu.VMEM((16,), jnp.float32),
        pltpu.VMEM((2 * 16 * CHUNK,), jnp.int32),
        pltpu.VMEM((2 * 16 * CHUNK,), jnp.float32),
        pltpu.VMEM((2, 8, CHUNK, C), jnp.float32),
        pltpu.VMEM((2, CHUNK, C), jnp.float32),
        pltpu.VMEM((32 * 32, C), jnp.float32),
        pltpu.VMEM((16 * 16, C), jnp.float32),
        pltpu.SemaphoreType.DMA,
        pltpu.SemaphoreType.DMA,
        pltpu.SemaphoreType.DMA,
        pltpu.SemaphoreType.DMA,
    ],
    compiler_params=pltpu.CompilerParams(
        needs_layout_passes=False, use_tc_tiling_on_sc=False),
)


@jax.jit
def kernel(points, node_uv_projection, featmap0, featmap1, featmap2, featmap3):
    pts_r = points.transpose(0, 1, 3, 2).reshape(NW, 3, N)
    proj_pad = jnp.pad(node_uv_projection.reshape(NW, 6), ((0, 0), (0, 10)))
    tabs = [fm.transpose(0, 2, 3, 1).reshape(G * r * r, C)
            for fm, r in zip((featmap0, featmap1, featmap2, featmap3), RES)]
    out = _sc_call(pts_r, proj_pad, *tabs)
    return out.reshape(B, G, N, C)
